# baseline (speedup 1.0000x reference)
import jax
import jax.numpy as jnp
from jax import lax
from jax.experimental import pallas as pl
from jax.experimental.pallas import tpu as pltpu
from jax.experimental.pallas import tpu_sc as plsc

NUM_CLASS = 1000
N_CLS_CTX = 4
CTX_DIM = 512
PREFIX_LEN = 6
SUFFIX_LEN = 67
SEQ_LEN = PREFIX_LEN + N_CLS_CTX + SUFFIX_LEN
BATCH = 1024

NC = 2
NS = 16
NW = NC * NS
BPW = BATCH // NW
GCHUNK = 8
NCHUNK = BPW // GCHUNK


def _patch(prompt_v, rows_v, j):
    for r in range(N_CLS_CTX):
        for c in range(CTX_DIM // 16):
            prompt_v[PREFIX_LEN + r, pl.ds(c * 16, 16)] = (
                rows_v[j, r, pl.ds(c * 16, 16)])


def _body(cls_hbm, idx_hbm, tmpl_hbm, out_hbm,
          idx_v, rows0, rows1, pA, pB, gsem, semA, semB):
    wid = lax.axis_index("s") * NC + lax.axis_index("c")
    base = wid * BPW

    pltpu.sync_copy(idx_hbm.at[pl.ds(base, BPW)], idx_v)

    rows = [rows0, rows1]
    gathers = [None] * NCHUNK
    gathers[0] = pltpu.async_copy(
        cls_hbm.at[idx_v.at[pl.ds(0, GCHUNK)]], rows[0], gsem)

    pltpu.sync_copy(tmpl_hbm, pA)
    pltpu.sync_copy(tmpl_hbm, pB)

    for h in range(NCHUNK):
        gathers[h].wait()
        if h + 1 < NCHUNK:
            gathers[h + 1] = pltpu.async_copy(
                cls_hbm.at[idx_v.at[pl.ds((h + 1) * GCHUNK, GCHUNK)]],
                rows[(h + 1) % 2], gsem)
        r = rows[h % 2]
        boff = base + h * GCHUNK

        def pair(i, carry):
            j = 2 * i

            def do_a():
                pltpu.make_async_copy(tmpl_hbm, pA, semA).wait()
            if h == 0:
                pl.when(i > 0)(do_a)
            else:
                do_a()
            _patch(pA, r, j)
            pltpu.async_copy(pA, out_hbm.at[boff + j], semA)

            def do_b():
                pltpu.make_async_copy(tmpl_hbm, pB, semB).wait()
            if h == 0:
                pl.when(i > 0)(do_b)
            else:
                do_b()
            _patch(pB, r, j + 1)
            pltpu.async_copy(pB, out_hbm.at[boff + j + 1], semB)
            return carry

        lax.fori_loop(0, GCHUNK // 2, pair, 0)

    pltpu.make_async_copy(tmpl_hbm, pA, semA).wait()
    pltpu.make_async_copy(tmpl_hbm, pB, semB).wait()


@jax.jit
def _prompt_learner(label, cls_ctx, tmpl):
    mesh = plsc.VectorSubcoreMesh(core_axis_name="c", subcore_axis_name="s")
    return pl.kernel(
        _body,
        out_type=jax.ShapeDtypeStruct((BATCH, SEQ_LEN, CTX_DIM), jnp.float32),
        mesh=mesh,
        scratch_types=[
            pltpu.VMEM((BPW,), jnp.int32),
            pltpu.VMEM((GCHUNK, N_CLS_CTX, CTX_DIM), jnp.float32),
            pltpu.VMEM((GCHUNK, N_CLS_CTX, CTX_DIM), jnp.float32),
            pltpu.VMEM((SEQ_LEN, CTX_DIM), jnp.float32),
            pltpu.VMEM((SEQ_LEN, CTX_DIM), jnp.float32),
            pltpu.SemaphoreType.DMA,
            pltpu.SemaphoreType.DMA,
            pltpu.SemaphoreType.DMA,
        ],
    )(cls_ctx, label, tmpl)


def kernel(label, cls_ctx, token_prefix, token_suffix):
    label = label.astype(jnp.int32)
    tmpl = jnp.concatenate(
        [token_prefix.reshape(PREFIX_LEN, CTX_DIM),
         jnp.zeros((N_CLS_CTX, CTX_DIM), jnp.float32),
         token_suffix.reshape(SUFFIX_LEN, CTX_DIM)], axis=0)
    return _prompt_learner(label, cls_ctx, tmpl)
